# baseline (speedup 1.0000x reference)
import jax
import jax.numpy as jnp
from jax import lax
from jax.experimental import pallas as pl
from jax.experimental.pallas import tpu as pltpu
from jax.experimental.pallas import tpu_sc as plsc

B = 8
P = 4096
S = 64
NC = 2
NS = 16
NW = NC * NS
PW = (B * P) // NW
CH = 128
NCH = PW // CH
TPC = CH // 16
OPIX = S * 128


def _sc_body(corr_hbm, gt_hbm, ref_hbm, ws_hbm, wc_hbm,
             gtx_v, gty_v, ref_v,
             e0_v, e1_v, e2_v, e3_v,
             q0_v, q1_v, q2_v, q3_v,
             cb0_v, cb1_v, cb2_v, cb3_v,
             os0_v, os1_v, os2_v, oc0_v, oc1_v, oc2_v,
             sem0, sem1, sem2, sem3):
    wid = lax.axis_index("s") * NC + lax.axis_index("c")
    b = wid // 4
    base = (wid % 4) * PW

    i0 = (wid % 4) * 16
    pltpu.sync_copy(gt_hbm.at[b, 0, pl.ds(i0, 16)], gtx_v)
    pltpu.sync_copy(gt_hbm.at[b, 1, pl.ds(i0, 16)], gty_v)
    pltpu.sync_copy(ref_hbm.at[b], ref_v)

    lanes = lax.iota(jnp.int32, 16)
    rbase0 = b * P + base

    def a_chunk(ch):
        for tt in range(TPC):
            off = ch * CH + tt * 16
            sl = pl.ds(off, 16)
            ir = ch * 2 + (tt >> 2)
            jsl = pl.ds((tt & 3) * 16, 16)
            gx = (gtx_v[ir, jsl] + 1.0) * 63.0 / 2.0
            gy = (gty_v[ir, jsl] + 1.0) * 63.0 / 2.0
            fx = gx.astype(jnp.int32)
            fy = gy.astype(jnp.int32)
            fxf = fx.astype(jnp.float32)
            fyf = fy.astype(jnp.float32)

            cx1 = jnp.minimum(fx + 1, S - 1)
            cy1 = jnp.minimum(fy + 1, S - 1)
            q00 = fy * S + fx
            q01 = fy * S + cx1
            q10 = cy1 * S + fx
            q11 = cy1 * S + cx1

            r = rbase0 + off + lanes
            ebase = (r >> 3) * 32768 + (r & 7) * 128
            e0_v[sl] = ebase + ((q00 >> 7) << 10) + (q00 & 127)
            e1_v[sl] = ebase + ((q01 >> 7) << 10) + (q01 & 127)
            e2_v[sl] = ebase + ((q10 >> 7) << 10) + (q10 & 127)
            e3_v[sl] = ebase + ((q11 >> 7) << 10) + (q11 & 127)
            q0_v[sl] = fy
            q1_v[sl] = fx
            q2_v[sl] = cy1
            q3_v[sl] = cx1

            wx0 = (fxf + 1.0) - gx
            wx1 = gx - fxf
            wy0 = (fyf + 1.0) - gy
            wy1 = gy - fyf
            mx = jnp.minimum((S - 1) - fx, 1).astype(jnp.float32)
            my = jnp.minimum((S - 1) - fy, 1).astype(jnp.float32)
            a0 = wy0 * wx0 * (mx * my)
            a1 = wy0 * wx1 * my
            a2 = wy1 * wx0 * mx
            a3 = wy1 * wx1

            z0 = jnp.zeros((16,), jnp.int32)
            z1 = z0 + 1
            z2 = z0 + 2
            r0c0 = plsc.load_gather(ref_v, [z0, fy, fx])
            r0c1 = plsc.load_gather(ref_v, [z1, fy, fx])
            r0c2 = plsc.load_gather(ref_v, [z2, fy, fx])
            r1c0 = plsc.load_gather(ref_v, [z0, fy, cx1])
            r1c1 = plsc.load_gather(ref_v, [z1, fy, cx1])
            r1c2 = plsc.load_gather(ref_v, [z2, fy, cx1])
            r2c0 = plsc.load_gather(ref_v, [z0, cy1, fx])
            r2c1 = plsc.load_gather(ref_v, [z1, cy1, fx])
            r2c2 = plsc.load_gather(ref_v, [z2, cy1, fx])
            r3c0 = plsc.load_gather(ref_v, [z0, cy1, cx1])
            r3c1 = plsc.load_gather(ref_v, [z1, cy1, cx1])
            r3c2 = plsc.load_gather(ref_v, [z2, cy1, cx1])

            osl = pl.ds((off >> 6) * 128 + (off & 63), 16)
            os0_v[osl] = a0 * r0c0 + a1 * r1c0 + a2 * r2c0 + a3 * r3c0
            os1_v[osl] = a0 * r0c1 + a1 * r1c1 + a2 * r2c1 + a3 * r3c1
            os2_v[osl] = a0 * r0c2 + a1 * r1c2 + a2 * r2c2 + a3 * r3c2

        csl = pl.ds(ch * CH, CH)
        pltpu.async_copy(corr_hbm.at[e0_v.at[csl]], cb0_v.at[csl], sem0)
        pltpu.async_copy(corr_hbm.at[e1_v.at[csl]], cb1_v.at[csl], sem1)
        pltpu.async_copy(corr_hbm.at[e2_v.at[csl]], cb2_v.at[csl], sem2)
        pltpu.async_copy(corr_hbm.at[e3_v.at[csl]], cb3_v.at[csl], sem3)

    def wait_chunk(ch):
        csl = pl.ds(ch * CH, CH)
        pltpu.make_async_copy(corr_hbm.at[e0_v.at[csl]], cb0_v.at[csl], sem0).wait()
        pltpu.make_async_copy(corr_hbm.at[e1_v.at[csl]], cb1_v.at[csl], sem1).wait()
        pltpu.make_async_copy(corr_hbm.at[e2_v.at[csl]], cb2_v.at[csl], sem2).wait()
        pltpu.make_async_copy(corr_hbm.at[e3_v.at[csl]], cb3_v.at[csl], sem3).wait()

    def c_chunk(ch):
        for tt in range(TPC):
            sl = pl.ds(ch * CH + tt * 16, 16)
            fy = q0_v[sl]
            fx = q1_v[sl]
            cy1 = q2_v[sl]
            cx1 = q3_v[sl]
            mx = jnp.minimum(cx1 - fx, 1).astype(jnp.float32)
            my = jnp.minimum(cy1 - fy, 1).astype(jnp.float32)

            c00 = cb0_v[sl] * (mx * my)
            c01 = cb1_v[sl] * my
            c10 = cb2_v[sl] * mx
            c11 = cb3_v[sl]

            z0 = jnp.zeros((16,), jnp.int32)
            z1 = z0 + 1
            z2 = z0 + 2
            r0c0 = plsc.load_gather(ref_v, [z0, fy, fx])
            r0c1 = plsc.load_gather(ref_v, [z1, fy, fx])
            r0c2 = plsc.load_gather(ref_v, [z2, fy, fx])
            r1c0 = plsc.load_gather(ref_v, [z0, fy, cx1])
            r1c1 = plsc.load_gather(ref_v, [z1, fy, cx1])
            r1c2 = plsc.load_gather(ref_v, [z2, fy, cx1])
            r2c0 = plsc.load_gather(ref_v, [z0, cy1, fx])
            r2c1 = plsc.load_gather(ref_v, [z1, cy1, fx])
            r2c2 = plsc.load_gather(ref_v, [z2, cy1, fx])
            r3c0 = plsc.load_gather(ref_v, [z0, cy1, cx1])
            r3c1 = plsc.load_gather(ref_v, [z1, cy1, cx1])
            r3c2 = plsc.load_gather(ref_v, [z2, cy1, cx1])

            osl = pl.ds((ch * 2 + (tt >> 2)) * 128 + ((tt & 3) << 4), 16)
            oc0_v[osl] = c00 * r0c0 + c01 * r1c0 + c10 * r2c0 + c11 * r3c0
            oc1_v[osl] = c00 * r0c1 + c01 * r1c1 + c10 * r2c1 + c11 * r3c1
            oc2_v[osl] = c00 * r0c2 + c01 * r1c2 + c10 * r2c2 + c11 * r3c2

    a_chunk(0)

    def steady(ch, carry):
        a_chunk(ch)
        wait_chunk(ch - 1)
        c_chunk(ch - 1)
        return carry

    lax.fori_loop(1, NCH, steady, 0)
    wait_chunk(NCH - 1)
    c_chunk(NCH - 1)

    obase = (wid % 4) * 2048
    pltpu.sync_copy(os0_v, ws_hbm.at[pl.ds((b * 3 + 0) * OPIX + obase, 2048)])
    pltpu.sync_copy(os1_v, ws_hbm.at[pl.ds((b * 3 + 1) * OPIX + obase, 2048)])
    pltpu.sync_copy(os2_v, ws_hbm.at[pl.ds((b * 3 + 2) * OPIX + obase, 2048)])
    pltpu.sync_copy(oc0_v, wc_hbm.at[pl.ds((b * 3 + 0) * OPIX + obase, 2048)])
    pltpu.sync_copy(oc1_v, wc_hbm.at[pl.ds((b * 3 + 1) * OPIX + obase, 2048)])
    pltpu.sync_copy(oc2_v, wc_hbm.at[pl.ds((b * 3 + 2) * OPIX + obase, 2048)])


def _build_sc_call():
    mesh = plsc.VectorSubcoreMesh(core_axis_name="c", subcore_axis_name="s",
                                  num_cores=NC, num_subcores=NS)
    out = jax.ShapeDtypeStruct((B * 3 * OPIX,), jnp.float32)
    return pl.kernel(
        _sc_body,
        out_type=(out, out),
        mesh=mesh,
        scratch_types=[
            pltpu.VMEM((16, S), jnp.float32),
            pltpu.VMEM((16, S), jnp.float32),
            pltpu.VMEM((3, S, S), jnp.float32),
            pltpu.VMEM((PW,), jnp.int32),
            pltpu.VMEM((PW,), jnp.int32),
            pltpu.VMEM((PW,), jnp.int32),
            pltpu.VMEM((PW,), jnp.int32),
            pltpu.VMEM((PW,), jnp.int32),
            pltpu.VMEM((PW,), jnp.int32),
            pltpu.VMEM((PW,), jnp.int32),
            pltpu.VMEM((PW,), jnp.int32),
            pltpu.VMEM((PW,), jnp.float32),
            pltpu.VMEM((PW,), jnp.float32),
            pltpu.VMEM((PW,), jnp.float32),
            pltpu.VMEM((PW,), jnp.float32),
            pltpu.VMEM((2048,), jnp.float32),
            pltpu.VMEM((2048,), jnp.float32),
            pltpu.VMEM((2048,), jnp.float32),
            pltpu.VMEM((2048,), jnp.float32),
            pltpu.VMEM((2048,), jnp.float32),
            pltpu.VMEM((2048,), jnp.float32),
            pltpu.SemaphoreType.DMA,
            pltpu.SemaphoreType.DMA,
            pltpu.SemaphoreType.DMA,
            pltpu.SemaphoreType.DMA,
        ],
        compiler_params=pltpu.CompilerParams(needs_layout_passes=False),
    )


def kernel(corr_m, gt_flow, vis_mask, scale_ref):
    del vis_mask
    corr_flat = (corr_m.reshape(B * P // 8, 8, P // 128, 128)
                 .transpose(0, 2, 1, 3)
                 .reshape(B * P * P))

    ws_p, wc_p = _build_sc_call()(corr_flat, gt_flow, scale_ref)
    warp_smpl = ws_p.reshape(B, 3, S, 128)[..., :S]
    warp_corr = wc_p.reshape(B, 3, S, 128)[..., :S]
    return (warp_smpl, warp_corr)


# speedup vs baseline: 1.0950x; 1.0950x over previous
import jax
import jax.numpy as jnp
from jax import lax
from jax.experimental import pallas as pl
from jax.experimental.pallas import tpu as pltpu
from jax.experimental.pallas import tpu_sc as plsc

B = 8
P = 4096
S = 64
NC = 2
NS = 16
NW = NC * NS
PW = (B * P) // NW
CH = 128
NCH = PW // CH
TPC = CH // 16
OPIX = S * 128


def _sc_body(corr_hbm, gt_hbm, ref_hbm, ws_hbm, wc_hbm,
             gtx_v, gty_v, ref_v,
             e0_v, e1_v, e2_v, e3_v,
             q0_v, q1_v, q2_v, q3_v,
             cb0_v, cb1_v, cb2_v, cb3_v,
             os0_v, os1_v, os2_v, oc0_v, oc1_v, oc2_v,
             sem0, sem1, sem2, sem3):
    wid = lax.axis_index("s") * NC + lax.axis_index("c")
    b = wid // 4
    base = (wid % 4) * PW

    i0 = (wid % 4) * 16
    pltpu.sync_copy(gt_hbm.at[b, 0, pl.ds(i0, 16)], gtx_v)
    pltpu.sync_copy(gt_hbm.at[b, 1, pl.ds(i0, 16)], gty_v)
    pltpu.sync_copy(ref_hbm.at[b], ref_v)

    lanes = lax.iota(jnp.int32, 16)
    rbase0 = b * P + base

    def phase_a(ch, carry):
        def a_iter(tt, carry2):
            off = ch * CH + tt * 16
            sl = pl.ds(off, 16)
            ir = ch * 2 + (tt >> 2)
            jsl = pl.ds((tt & 3) * 16, 16)
            gx = (gtx_v[ir, jsl] + 1.0) * 63.0 / 2.0
            gy = (gty_v[ir, jsl] + 1.0) * 63.0 / 2.0
            fx = gx.astype(jnp.int32)
            fy = gy.astype(jnp.int32)
            fxf = fx.astype(jnp.float32)
            fyf = fy.astype(jnp.float32)

            cx1 = jnp.minimum(fx + 1, S - 1)
            cy1 = jnp.minimum(fy + 1, S - 1)
            q00 = fy * S + fx
            q01 = fy * S + cx1
            q10 = cy1 * S + fx
            q11 = cy1 * S + cx1

            r = rbase0 + off + lanes
            ebase = (r >> 3) * 32768 + (r & 7) * 128
            e0_v[sl] = ebase + ((q00 >> 7) << 10) + (q00 & 127)
            e1_v[sl] = ebase + ((q01 >> 7) << 10) + (q01 & 127)
            e2_v[sl] = ebase + ((q10 >> 7) << 10) + (q10 & 127)
            e3_v[sl] = ebase + ((q11 >> 7) << 10) + (q11 & 127)
            q0_v[sl] = fy
            q1_v[sl] = fx
            q2_v[sl] = cy1
            q3_v[sl] = cx1

            wx0 = (fxf + 1.0) - gx
            wx1 = gx - fxf
            wy0 = (fyf + 1.0) - gy
            wy1 = gy - fyf
            mx = jnp.minimum((S - 1) - fx, 1).astype(jnp.float32)
            my = jnp.minimum((S - 1) - fy, 1).astype(jnp.float32)
            a0 = wy0 * wx0 * (mx * my)
            a1 = wy0 * wx1 * my
            a2 = wy1 * wx0 * mx
            a3 = wy1 * wx1

            z0 = jnp.zeros((16,), jnp.int32)
            z1 = z0 + 1
            z2 = z0 + 2
            r0c0 = plsc.load_gather(ref_v, [z0, fy, fx])
            r0c1 = plsc.load_gather(ref_v, [z1, fy, fx])
            r0c2 = plsc.load_gather(ref_v, [z2, fy, fx])
            r1c0 = plsc.load_gather(ref_v, [z0, fy, cx1])
            r1c1 = plsc.load_gather(ref_v, [z1, fy, cx1])
            r1c2 = plsc.load_gather(ref_v, [z2, fy, cx1])
            r2c0 = plsc.load_gather(ref_v, [z0, cy1, fx])
            r2c1 = plsc.load_gather(ref_v, [z1, cy1, fx])
            r2c2 = plsc.load_gather(ref_v, [z2, cy1, fx])
            r3c0 = plsc.load_gather(ref_v, [z0, cy1, cx1])
            r3c1 = plsc.load_gather(ref_v, [z1, cy1, cx1])
            r3c2 = plsc.load_gather(ref_v, [z2, cy1, cx1])

            osl = pl.ds((off >> 6) * 128 + (off & 63), 16)
            os0_v[osl] = a0 * r0c0 + a1 * r1c0 + a2 * r2c0 + a3 * r3c0
            os1_v[osl] = a0 * r0c1 + a1 * r1c1 + a2 * r2c1 + a3 * r3c1
            os2_v[osl] = a0 * r0c2 + a1 * r1c2 + a2 * r2c2 + a3 * r3c2
            return carry2

        lax.fori_loop(0, TPC, a_iter, 0)

        csl = pl.ds(ch * CH, CH)
        pltpu.async_copy(corr_hbm.at[e0_v.at[csl]], cb0_v.at[csl], sem0)
        pltpu.async_copy(corr_hbm.at[e1_v.at[csl]], cb1_v.at[csl], sem1)
        pltpu.async_copy(corr_hbm.at[e2_v.at[csl]], cb2_v.at[csl], sem2)
        pltpu.async_copy(corr_hbm.at[e3_v.at[csl]], cb3_v.at[csl], sem3)
        return carry

    lax.fori_loop(0, NCH, phase_a, 0)

    def phase_b(ch, carry):
        csl = pl.ds(ch * CH, CH)
        pltpu.make_async_copy(corr_hbm.at[e0_v.at[csl]], cb0_v.at[csl], sem0).wait()
        pltpu.make_async_copy(corr_hbm.at[e1_v.at[csl]], cb1_v.at[csl], sem1).wait()
        pltpu.make_async_copy(corr_hbm.at[e2_v.at[csl]], cb2_v.at[csl], sem2).wait()
        pltpu.make_async_copy(corr_hbm.at[e3_v.at[csl]], cb3_v.at[csl], sem3).wait()
        return carry

    lax.fori_loop(0, NCH, phase_b, 0)

    def phase_c(t, carry):
        sl = pl.ds(t * 16, 16)
        fy = q0_v[sl]
        fx = q1_v[sl]
        cy1 = q2_v[sl]
        cx1 = q3_v[sl]
        mx = jnp.minimum(cx1 - fx, 1).astype(jnp.float32)
        my = jnp.minimum(cy1 - fy, 1).astype(jnp.float32)

        c00 = cb0_v[sl] * (mx * my)
        c01 = cb1_v[sl] * my
        c10 = cb2_v[sl] * mx
        c11 = cb3_v[sl]

        z0 = jnp.zeros((16,), jnp.int32)
        z1 = z0 + 1
        z2 = z0 + 2
        r0c0 = plsc.load_gather(ref_v, [z0, fy, fx])
        r0c1 = plsc.load_gather(ref_v, [z1, fy, fx])
        r0c2 = plsc.load_gather(ref_v, [z2, fy, fx])
        r1c0 = plsc.load_gather(ref_v, [z0, fy, cx1])
        r1c1 = plsc.load_gather(ref_v, [z1, fy, cx1])
        r1c2 = plsc.load_gather(ref_v, [z2, fy, cx1])
        r2c0 = plsc.load_gather(ref_v, [z0, cy1, fx])
        r2c1 = plsc.load_gather(ref_v, [z1, cy1, fx])
        r2c2 = plsc.load_gather(ref_v, [z2, cy1, fx])
        r3c0 = plsc.load_gather(ref_v, [z0, cy1, cx1])
        r3c1 = plsc.load_gather(ref_v, [z1, cy1, cx1])
        r3c2 = plsc.load_gather(ref_v, [z2, cy1, cx1])

        osl = pl.ds((t >> 2) * 128 + ((t & 3) << 4), 16)
        oc0_v[osl] = c00 * r0c0 + c01 * r1c0 + c10 * r2c0 + c11 * r3c0
        oc1_v[osl] = c00 * r0c1 + c01 * r1c1 + c10 * r2c1 + c11 * r3c1
        oc2_v[osl] = c00 * r0c2 + c01 * r1c2 + c10 * r2c2 + c11 * r3c2
        return carry

    lax.fori_loop(0, PW // 16, phase_c, 0)

    obase = (wid % 4) * 2048
    pltpu.sync_copy(os0_v, ws_hbm.at[pl.ds((b * 3 + 0) * OPIX + obase, 2048)])
    pltpu.sync_copy(os1_v, ws_hbm.at[pl.ds((b * 3 + 1) * OPIX + obase, 2048)])
    pltpu.sync_copy(os2_v, ws_hbm.at[pl.ds((b * 3 + 2) * OPIX + obase, 2048)])
    pltpu.sync_copy(oc0_v, wc_hbm.at[pl.ds((b * 3 + 0) * OPIX + obase, 2048)])
    pltpu.sync_copy(oc1_v, wc_hbm.at[pl.ds((b * 3 + 1) * OPIX + obase, 2048)])
    pltpu.sync_copy(oc2_v, wc_hbm.at[pl.ds((b * 3 + 2) * OPIX + obase, 2048)])


def _build_sc_call():
    mesh = plsc.VectorSubcoreMesh(core_axis_name="c", subcore_axis_name="s",
                                  num_cores=NC, num_subcores=NS)
    out = jax.ShapeDtypeStruct((B * 3 * OPIX,), jnp.float32)
    return pl.kernel(
        _sc_body,
        out_type=(out, out),
        mesh=mesh,
        scratch_types=[
            pltpu.VMEM((16, S), jnp.float32),
            pltpu.VMEM((16, S), jnp.float32),
            pltpu.VMEM((3, S, S), jnp.float32),
            pltpu.VMEM((PW,), jnp.int32),
            pltpu.VMEM((PW,), jnp.int32),
            pltpu.VMEM((PW,), jnp.int32),
            pltpu.VMEM((PW,), jnp.int32),
            pltpu.VMEM((PW,), jnp.int32),
            pltpu.VMEM((PW,), jnp.int32),
            pltpu.VMEM((PW,), jnp.int32),
            pltpu.VMEM((PW,), jnp.int32),
            pltpu.VMEM((PW,), jnp.float32),
            pltpu.VMEM((PW,), jnp.float32),
            pltpu.VMEM((PW,), jnp.float32),
            pltpu.VMEM((PW,), jnp.float32),
            pltpu.VMEM((2048,), jnp.float32),
            pltpu.VMEM((2048,), jnp.float32),
            pltpu.VMEM((2048,), jnp.float32),
            pltpu.VMEM((2048,), jnp.float32),
            pltpu.VMEM((2048,), jnp.float32),
            pltpu.VMEM((2048,), jnp.float32),
            pltpu.SemaphoreType.DMA,
            pltpu.SemaphoreType.DMA,
            pltpu.SemaphoreType.DMA,
            pltpu.SemaphoreType.DMA,
        ],
        compiler_params=pltpu.CompilerParams(needs_layout_passes=False),
    )


def kernel(corr_m, gt_flow, vis_mask, scale_ref):
    del vis_mask
    corr_flat = (corr_m.reshape(B * P // 8, 8, P // 128, 128)
                 .transpose(0, 2, 1, 3)
                 .reshape(B * P * P))

    ws_p, wc_p = _build_sc_call()(corr_flat, gt_flow, scale_ref)
    warp_smpl = ws_p.reshape(B, 3, S, 128)[..., :S]
    warp_corr = wc_p.reshape(B, 3, S, 128)[..., :S]
    return (warp_smpl, warp_corr)


# speedup vs baseline: 1.1462x; 1.0468x over previous
import jax
import jax.numpy as jnp
from jax import lax
from jax.experimental import pallas as pl
from jax.experimental.pallas import tpu as pltpu
from jax.experimental.pallas import tpu_sc as plsc

B = 8
P = 4096
S = 64
NC = 2
NS = 16
NW = NC * NS
PW = (B * P) // NW
CH = 128
NCH = PW // CH
TPC = CH // 16
OPIX = S * 128


def _sc_body(corr_hbm, gt_hbm, ref_hbm, ws_hbm, wc_hbm,
             gtx_v, gty_v, ref_v,
             e0_v, e1_v, e2_v, e3_v,
             q0_v, q1_v, q2_v, q3_v,
             cb0_v, cb1_v, cb2_v, cb3_v,
             os0_v, os1_v, os2_v, oc0_v, oc1_v, oc2_v,
             sem0, sem1, sem2, sem3):
    wid = lax.axis_index("s") * NC + lax.axis_index("c")
    b = wid // 4
    base = (wid % 4) * PW

    i0 = (wid % 4) * 16
    d_gtx = pltpu.async_copy(gt_hbm.at[b, 0, pl.ds(i0, 16)], gtx_v, sem0)
    d_gty = pltpu.async_copy(gt_hbm.at[b, 1, pl.ds(i0, 16)], gty_v, sem1)
    d_ref = pltpu.async_copy(ref_hbm.at[b], ref_v, sem2)
    d_gtx.wait()
    d_gty.wait()
    d_ref.wait()

    lanes = lax.iota(jnp.int32, 16)
    rbase0 = b * P + base

    def phase_a(ch, carry):
        def a_iter(tt, carry2):
            off = ch * CH + tt * 16
            sl = pl.ds(off, 16)
            ir = ch * 2 + (tt >> 2)
            jsl = pl.ds((tt & 3) * 16, 16)
            gx = (gtx_v[ir, jsl] + 1.0) * 63.0 / 2.0
            gy = (gty_v[ir, jsl] + 1.0) * 63.0 / 2.0
            fx = gx.astype(jnp.int32)
            fy = gy.astype(jnp.int32)
            fxf = fx.astype(jnp.float32)
            fyf = fy.astype(jnp.float32)

            cx1 = jnp.minimum(fx + 1, S - 1)
            cy1 = jnp.minimum(fy + 1, S - 1)
            q00 = fy * S + fx
            q01 = fy * S + cx1
            q10 = cy1 * S + fx
            q11 = cy1 * S + cx1

            r = rbase0 + off + lanes
            ebase = (r >> 3) * 32768 + (r & 7) * 128
            e0_v[sl] = ebase + ((q00 >> 7) << 10) + (q00 & 127)
            e1_v[sl] = ebase + ((q01 >> 7) << 10) + (q01 & 127)
            e2_v[sl] = ebase + ((q10 >> 7) << 10) + (q10 & 127)
            e3_v[sl] = ebase + ((q11 >> 7) << 10) + (q11 & 127)
            q0_v[sl] = fy
            q1_v[sl] = fx
            q2_v[sl] = cy1
            q3_v[sl] = cx1

            wx0 = (fxf + 1.0) - gx
            wx1 = gx - fxf
            wy0 = (fyf + 1.0) - gy
            wy1 = gy - fyf
            mx = jnp.minimum((S - 1) - fx, 1).astype(jnp.float32)
            my = jnp.minimum((S - 1) - fy, 1).astype(jnp.float32)
            a0 = wy0 * wx0 * (mx * my)
            a1 = wy0 * wx1 * my
            a2 = wy1 * wx0 * mx
            a3 = wy1 * wx1

            z0 = jnp.zeros((16,), jnp.int32)
            z1 = z0 + 1
            z2 = z0 + 2
            r0c0 = plsc.load_gather(ref_v, [z0, fy, fx])
            r0c1 = plsc.load_gather(ref_v, [z1, fy, fx])
            r0c2 = plsc.load_gather(ref_v, [z2, fy, fx])
            r1c0 = plsc.load_gather(ref_v, [z0, fy, cx1])
            r1c1 = plsc.load_gather(ref_v, [z1, fy, cx1])
            r1c2 = plsc.load_gather(ref_v, [z2, fy, cx1])
            r2c0 = plsc.load_gather(ref_v, [z0, cy1, fx])
            r2c1 = plsc.load_gather(ref_v, [z1, cy1, fx])
            r2c2 = plsc.load_gather(ref_v, [z2, cy1, fx])
            r3c0 = plsc.load_gather(ref_v, [z0, cy1, cx1])
            r3c1 = plsc.load_gather(ref_v, [z1, cy1, cx1])
            r3c2 = plsc.load_gather(ref_v, [z2, cy1, cx1])

            osl = pl.ds((off >> 6) * 128 + (off & 63), 16)
            os0_v[osl] = a0 * r0c0 + a1 * r1c0 + a2 * r2c0 + a3 * r3c0
            os1_v[osl] = a0 * r0c1 + a1 * r1c1 + a2 * r2c1 + a3 * r3c1
            os2_v[osl] = a0 * r0c2 + a1 * r1c2 + a2 * r2c2 + a3 * r3c2
            return carry2

        lax.fori_loop(0, TPC, a_iter, 0)

        csl = pl.ds(ch * CH, CH)
        pltpu.async_copy(corr_hbm.at[e0_v.at[csl]], cb0_v.at[csl], sem0)
        pltpu.async_copy(corr_hbm.at[e1_v.at[csl]], cb1_v.at[csl], sem1)
        pltpu.async_copy(corr_hbm.at[e2_v.at[csl]], cb2_v.at[csl], sem2)
        pltpu.async_copy(corr_hbm.at[e3_v.at[csl]], cb3_v.at[csl], sem3)
        return carry

    lax.fori_loop(0, NCH, phase_a, 0)

    def phase_b(ch, carry):
        csl = pl.ds(ch * CH, CH)
        pltpu.make_async_copy(corr_hbm.at[e0_v.at[csl]], cb0_v.at[csl], sem0).wait()
        pltpu.make_async_copy(corr_hbm.at[e1_v.at[csl]], cb1_v.at[csl], sem1).wait()
        pltpu.make_async_copy(corr_hbm.at[e2_v.at[csl]], cb2_v.at[csl], sem2).wait()
        pltpu.make_async_copy(corr_hbm.at[e3_v.at[csl]], cb3_v.at[csl], sem3).wait()
        return carry

    lax.fori_loop(0, NCH, phase_b, 0)

    def phase_c(t, carry):
        sl = pl.ds(t * 16, 16)
        fy = q0_v[sl]
        fx = q1_v[sl]
        cy1 = q2_v[sl]
        cx1 = q3_v[sl]
        mx = jnp.minimum(cx1 - fx, 1).astype(jnp.float32)
        my = jnp.minimum(cy1 - fy, 1).astype(jnp.float32)

        c00 = cb0_v[sl] * (mx * my)
        c01 = cb1_v[sl] * my
        c10 = cb2_v[sl] * mx
        c11 = cb3_v[sl]

        z0 = jnp.zeros((16,), jnp.int32)
        z1 = z0 + 1
        z2 = z0 + 2
        r0c0 = plsc.load_gather(ref_v, [z0, fy, fx])
        r0c1 = plsc.load_gather(ref_v, [z1, fy, fx])
        r0c2 = plsc.load_gather(ref_v, [z2, fy, fx])
        r1c0 = plsc.load_gather(ref_v, [z0, fy, cx1])
        r1c1 = plsc.load_gather(ref_v, [z1, fy, cx1])
        r1c2 = plsc.load_gather(ref_v, [z2, fy, cx1])
        r2c0 = plsc.load_gather(ref_v, [z0, cy1, fx])
        r2c1 = plsc.load_gather(ref_v, [z1, cy1, fx])
        r2c2 = plsc.load_gather(ref_v, [z2, cy1, fx])
        r3c0 = plsc.load_gather(ref_v, [z0, cy1, cx1])
        r3c1 = plsc.load_gather(ref_v, [z1, cy1, cx1])
        r3c2 = plsc.load_gather(ref_v, [z2, cy1, cx1])

        osl = pl.ds((t >> 2) * 128 + ((t & 3) << 4), 16)
        oc0_v[osl] = c00 * r0c0 + c01 * r1c0 + c10 * r2c0 + c11 * r3c0
        oc1_v[osl] = c00 * r0c1 + c01 * r1c1 + c10 * r2c1 + c11 * r3c1
        oc2_v[osl] = c00 * r0c2 + c01 * r1c2 + c10 * r2c2 + c11 * r3c2
        return carry

    lax.fori_loop(0, PW // 16, phase_c, 0)

    obase = (wid % 4) * 2048
    w0 = pltpu.async_copy(os0_v, ws_hbm.at[pl.ds((b * 3 + 0) * OPIX + obase, 2048)], sem0)
    w1 = pltpu.async_copy(os1_v, ws_hbm.at[pl.ds((b * 3 + 1) * OPIX + obase, 2048)], sem1)
    w2 = pltpu.async_copy(os2_v, ws_hbm.at[pl.ds((b * 3 + 2) * OPIX + obase, 2048)], sem2)
    w3 = pltpu.async_copy(oc0_v, wc_hbm.at[pl.ds((b * 3 + 0) * OPIX + obase, 2048)], sem3)
    w4 = pltpu.async_copy(oc1_v, wc_hbm.at[pl.ds((b * 3 + 1) * OPIX + obase, 2048)], sem0)
    w5 = pltpu.async_copy(oc2_v, wc_hbm.at[pl.ds((b * 3 + 2) * OPIX + obase, 2048)], sem1)
    w0.wait()
    w1.wait()
    w2.wait()
    w3.wait()
    w4.wait()
    w5.wait()


def _build_sc_call():
    mesh = plsc.VectorSubcoreMesh(core_axis_name="c", subcore_axis_name="s",
                                  num_cores=NC, num_subcores=NS)
    out = jax.ShapeDtypeStruct((B * 3 * OPIX,), jnp.float32)
    return pl.kernel(
        _sc_body,
        out_type=(out, out),
        mesh=mesh,
        scratch_types=[
            pltpu.VMEM((16, S), jnp.float32),
            pltpu.VMEM((16, S), jnp.float32),
            pltpu.VMEM((3, S, S), jnp.float32),
            pltpu.VMEM((PW,), jnp.int32),
            pltpu.VMEM((PW,), jnp.int32),
            pltpu.VMEM((PW,), jnp.int32),
            pltpu.VMEM((PW,), jnp.int32),
            pltpu.VMEM((PW,), jnp.int32),
            pltpu.VMEM((PW,), jnp.int32),
            pltpu.VMEM((PW,), jnp.int32),
            pltpu.VMEM((PW,), jnp.int32),
            pltpu.VMEM((PW,), jnp.float32),
            pltpu.VMEM((PW,), jnp.float32),
            pltpu.VMEM((PW,), jnp.float32),
            pltpu.VMEM((PW,), jnp.float32),
            pltpu.VMEM((2048,), jnp.float32),
            pltpu.VMEM((2048,), jnp.float32),
            pltpu.VMEM((2048,), jnp.float32),
            pltpu.VMEM((2048,), jnp.float32),
            pltpu.VMEM((2048,), jnp.float32),
            pltpu.VMEM((2048,), jnp.float32),
            pltpu.SemaphoreType.DMA,
            pltpu.SemaphoreType.DMA,
            pltpu.SemaphoreType.DMA,
            pltpu.SemaphoreType.DMA,
        ],
        compiler_params=pltpu.CompilerParams(needs_layout_passes=False),
    )


def kernel(corr_m, gt_flow, vis_mask, scale_ref):
    del vis_mask
    corr_flat = (corr_m.reshape(B * P // 8, 8, P // 128, 128)
                 .transpose(0, 2, 1, 3)
                 .reshape(B * P * P))

    ws_p, wc_p = _build_sc_call()(corr_flat, gt_flow, scale_ref)
    warp_smpl = ws_p.reshape(B, 3, S, 128)[..., :S]
    warp_corr = wc_p.reshape(B, 3, S, 128)[..., :S]
    return (warp_smpl, warp_corr)


# speedup vs baseline: 1.1505x; 1.0037x over previous
import jax
import jax.numpy as jnp
from jax import lax
from jax.experimental import pallas as pl
from jax.experimental.pallas import tpu as pltpu
from jax.experimental.pallas import tpu_sc as plsc

B = 8
P = 4096
S = 64
NC = 2
NS = 16
NW = NC * NS
PW = (B * P) // NW
CH = 128
NCH = PW // CH
TPC = CH // 16
OPIX = S * 128


def _sc_body(corr_hbm, gt_hbm, ref_hbm, ws_hbm, wc_hbm,
             gtx_v, gty_v, ref_v,
             e0_v, e1_v, e2_v, e3_v,
             q0_v, q1_v, q2_v, q3_v,
             cb0_v, cb1_v, cb2_v, cb3_v,
             os0_v, os1_v, os2_v, oc0_v, oc1_v, oc2_v,
             sem0, sem1, sem2, sem3):
    wid = lax.axis_index("s") * NC + lax.axis_index("c")
    b = wid // 4
    base = (wid % 4) * PW

    i0 = (wid % 4) * 16
    d_gtx = pltpu.async_copy(gt_hbm.at[b, 0, pl.ds(i0, 16)], gtx_v, sem0)
    d_gty = pltpu.async_copy(gt_hbm.at[b, 1, pl.ds(i0, 16)], gty_v, sem1)
    d_r0 = pltpu.async_copy(ref_hbm.at[b, 0], ref_v.at[pl.ds(0, S)], sem2)
    d_r1 = pltpu.async_copy(ref_hbm.at[b, 1], ref_v.at[pl.ds(S, S)], sem3)
    d_r2 = pltpu.async_copy(ref_hbm.at[b, 2], ref_v.at[pl.ds(2 * S, S)], sem2)
    d_gtx.wait()
    d_gty.wait()
    d_r0.wait()
    d_r1.wait()
    d_r2.wait()

    lanes = lax.iota(jnp.int32, 16)
    rbase0 = b * P + base
    eb_lanes = ((lanes >> 3) << 15) + ((lanes & 7) << 7)

    def phase_a(ch, carry):
        def a_iter(tt, carry2):
            off = ch * CH + tt * 16
            sl = pl.ds(off, 16)
            ir = ch * 2 + (tt >> 2)
            jsl = pl.ds((tt & 3) * 16, 16)
            gx = (gtx_v[ir, jsl] + 1.0) * 63.0 / 2.0
            gy = (gty_v[ir, jsl] + 1.0) * 63.0 / 2.0
            fx = gx.astype(jnp.int32)
            fy = gy.astype(jnp.int32)
            fxf = fx.astype(jnp.float32)
            fyf = fy.astype(jnp.float32)

            cx1 = jnp.minimum(fx + 1, S - 1)
            cy1 = jnp.minimum(fy + 1, S - 1)
            dx = cx1 - fx
            dy = cy1 - fy

            ebase = ((rbase0 + off) >> 3) * 32768 + eb_lanes + fx
            e0 = ebase + ((fy >> 1) << 10) + ((fy & 1) << 6)
            e2 = ebase + ((cy1 >> 1) << 10) + ((cy1 & 1) << 6)
            e0_v[sl] = e0
            e1_v[sl] = e0 + dx
            e2_v[sl] = e2
            e3_v[sl] = e2 + dx
            q0_v[sl] = fy
            q1_v[sl] = fx
            q2_v[sl] = cy1
            q3_v[sl] = cx1

            wx0 = (fxf + 1.0) - gx
            wx1 = gx - fxf
            wy0 = (fyf + 1.0) - gy
            wy1 = gy - fyf
            mx = dx.astype(jnp.float32)
            my = dy.astype(jnp.float32)
            a0 = wy0 * wx0 * (mx * my)
            a1 = wy0 * wx1 * my
            a2 = wy1 * wx0 * mx
            a3 = wy1 * wx1

            fyb = fy + S
            fyc = fy + 2 * S
            cyb = cy1 + S
            cyc = cy1 + 2 * S
            r0c0 = plsc.load_gather(ref_v, [fy, fx])
            r0c1 = plsc.load_gather(ref_v, [fyb, fx])
            r0c2 = plsc.load_gather(ref_v, [fyc, fx])
            r1c0 = plsc.load_gather(ref_v, [fy, cx1])
            r1c1 = plsc.load_gather(ref_v, [fyb, cx1])
            r1c2 = plsc.load_gather(ref_v, [fyc, cx1])
            r2c0 = plsc.load_gather(ref_v, [cy1, fx])
            r2c1 = plsc.load_gather(ref_v, [cyb, fx])
            r2c2 = plsc.load_gather(ref_v, [cyc, fx])
            r3c0 = plsc.load_gather(ref_v, [cy1, cx1])
            r3c1 = plsc.load_gather(ref_v, [cyb, cx1])
            r3c2 = plsc.load_gather(ref_v, [cyc, cx1])

            osl = pl.ds((off >> 6) * 128 + (off & 63), 16)
            os0_v[osl] = a0 * r0c0 + a1 * r1c0 + a2 * r2c0 + a3 * r3c0
            os1_v[osl] = a0 * r0c1 + a1 * r1c1 + a2 * r2c1 + a3 * r3c1
            os2_v[osl] = a0 * r0c2 + a1 * r1c2 + a2 * r2c2 + a3 * r3c2
            return carry2

        lax.fori_loop(0, TPC, a_iter, 0)

        csl = pl.ds(ch * CH, CH)
        pltpu.async_copy(corr_hbm.at[e0_v.at[csl]], cb0_v.at[csl], sem0)
        pltpu.async_copy(corr_hbm.at[e1_v.at[csl]], cb1_v.at[csl], sem1)
        pltpu.async_copy(corr_hbm.at[e2_v.at[csl]], cb2_v.at[csl], sem2)
        pltpu.async_copy(corr_hbm.at[e3_v.at[csl]], cb3_v.at[csl], sem3)
        return carry

    lax.fori_loop(0, NCH, phase_a, 0)

    def phase_b(ch, carry):
        csl = pl.ds(ch * CH, CH)
        pltpu.make_async_copy(corr_hbm.at[e0_v.at[csl]], cb0_v.at[csl], sem0).wait()
        pltpu.make_async_copy(corr_hbm.at[e1_v.at[csl]], cb1_v.at[csl], sem1).wait()
        pltpu.make_async_copy(corr_hbm.at[e2_v.at[csl]], cb2_v.at[csl], sem2).wait()
        pltpu.make_async_copy(corr_hbm.at[e3_v.at[csl]], cb3_v.at[csl], sem3).wait()
        return carry

    lax.fori_loop(0, NCH, phase_b, 0)

    def phase_c(t, carry):
        sl = pl.ds(t * 16, 16)
        fy = q0_v[sl]
        fx = q1_v[sl]
        cy1 = q2_v[sl]
        cx1 = q3_v[sl]
        mx = (cx1 - fx).astype(jnp.float32)
        my = (cy1 - fy).astype(jnp.float32)

        c00 = cb0_v[sl] * (mx * my)
        c01 = cb1_v[sl] * my
        c10 = cb2_v[sl] * mx
        c11 = cb3_v[sl]

        fyb = fy + S
        fyc = fy + 2 * S
        cyb = cy1 + S
        cyc = cy1 + 2 * S
        r0c0 = plsc.load_gather(ref_v, [fy, fx])
        r0c1 = plsc.load_gather(ref_v, [fyb, fx])
        r0c2 = plsc.load_gather(ref_v, [fyc, fx])
        r1c0 = plsc.load_gather(ref_v, [fy, cx1])
        r1c1 = plsc.load_gather(ref_v, [fyb, cx1])
        r1c2 = plsc.load_gather(ref_v, [fyc, cx1])
        r2c0 = plsc.load_gather(ref_v, [cy1, fx])
        r2c1 = plsc.load_gather(ref_v, [cyb, fx])
        r2c2 = plsc.load_gather(ref_v, [cyc, fx])
        r3c0 = plsc.load_gather(ref_v, [cy1, cx1])
        r3c1 = plsc.load_gather(ref_v, [cyb, cx1])
        r3c2 = plsc.load_gather(ref_v, [cyc, cx1])

        osl = pl.ds((t >> 2) * 128 + ((t & 3) << 4), 16)
        oc0_v[osl] = c00 * r0c0 + c01 * r1c0 + c10 * r2c0 + c11 * r3c0
        oc1_v[osl] = c00 * r0c1 + c01 * r1c1 + c10 * r2c1 + c11 * r3c1
        oc2_v[osl] = c00 * r0c2 + c01 * r1c2 + c10 * r2c2 + c11 * r3c2
        return carry

    lax.fori_loop(0, PW // 16, phase_c, 0)

    obase = (wid % 4) * 2048
    w0 = pltpu.async_copy(os0_v, ws_hbm.at[pl.ds((b * 3 + 0) * OPIX + obase, 2048)], sem0)
    w1 = pltpu.async_copy(os1_v, ws_hbm.at[pl.ds((b * 3 + 1) * OPIX + obase, 2048)], sem1)
    w2 = pltpu.async_copy(os2_v, ws_hbm.at[pl.ds((b * 3 + 2) * OPIX + obase, 2048)], sem2)
    w3 = pltpu.async_copy(oc0_v, wc_hbm.at[pl.ds((b * 3 + 0) * OPIX + obase, 2048)], sem3)
    w4 = pltpu.async_copy(oc1_v, wc_hbm.at[pl.ds((b * 3 + 1) * OPIX + obase, 2048)], sem0)
    w5 = pltpu.async_copy(oc2_v, wc_hbm.at[pl.ds((b * 3 + 2) * OPIX + obase, 2048)], sem1)
    w0.wait()
    w1.wait()
    w2.wait()
    w3.wait()
    w4.wait()
    w5.wait()


def _build_sc_call():
    mesh = plsc.VectorSubcoreMesh(core_axis_name="c", subcore_axis_name="s",
                                  num_cores=NC, num_subcores=NS)
    out = jax.ShapeDtypeStruct((B * 3 * OPIX,), jnp.float32)
    return pl.kernel(
        _sc_body,
        out_type=(out, out),
        mesh=mesh,
        scratch_types=[
            pltpu.VMEM((16, S), jnp.float32),
            pltpu.VMEM((16, S), jnp.float32),
            pltpu.VMEM((3 * S, S), jnp.float32),
            pltpu.VMEM((PW,), jnp.int32),
            pltpu.VMEM((PW,), jnp.int32),
            pltpu.VMEM((PW,), jnp.int32),
            pltpu.VMEM((PW,), jnp.int32),
            pltpu.VMEM((PW,), jnp.int32),
            pltpu.VMEM((PW,), jnp.int32),
            pltpu.VMEM((PW,), jnp.int32),
            pltpu.VMEM((PW,), jnp.int32),
            pltpu.VMEM((PW,), jnp.float32),
            pltpu.VMEM((PW,), jnp.float32),
            pltpu.VMEM((PW,), jnp.float32),
            pltpu.VMEM((PW,), jnp.float32),
            pltpu.VMEM((2048,), jnp.float32),
            pltpu.VMEM((2048,), jnp.float32),
            pltpu.VMEM((2048,), jnp.float32),
            pltpu.VMEM((2048,), jnp.float32),
            pltpu.VMEM((2048,), jnp.float32),
            pltpu.VMEM((2048,), jnp.float32),
            pltpu.SemaphoreType.DMA,
            pltpu.SemaphoreType.DMA,
            pltpu.SemaphoreType.DMA,
            pltpu.SemaphoreType.DMA,
        ],
        compiler_params=pltpu.CompilerParams(needs_layout_passes=False),
    )


def kernel(corr_m, gt_flow, vis_mask, scale_ref):
    del vis_mask
    corr_flat = (corr_m.reshape(B * P // 8, 8, P // 128, 128)
                 .transpose(0, 2, 1, 3)
                 .reshape(B * P * P))

    ws_p, wc_p = _build_sc_call()(corr_flat, gt_flow, scale_ref)
    warp_smpl = ws_p.reshape(B, 3, S, 128)[..., :S]
    warp_corr = wc_p.reshape(B, 3, S, 128)[..., :S]
    return (warp_smpl, warp_corr)


# speedup vs baseline: 1.1681x; 1.0153x over previous
import jax
import jax.numpy as jnp
from jax import lax
from jax.experimental import pallas as pl
from jax.experimental.pallas import tpu as pltpu
from jax.experimental.pallas import tpu_sc as plsc

B = 8
P = 4096
S = 64
NC = 2
NS = 16
NW = NC * NS
PW = (B * P) // NW
CH = 128
NCH = PW // CH
TPC = CH // 16
OPIX = S * 128


def _sc_body(corr_hbm, gt_hbm, ref_hbm, ws_hbm, wc_hbm,
             gtx_v, gty_v, ref_v,
             e0_v, e1_v, e2_v, e3_v,
             q0_v, q1_v, q2_v, q3_v,
             cb0_v, cb1_v, cb2_v, cb3_v,
             os0_v, os1_v, os2_v, oc0_v, oc1_v, oc2_v,
             sem0, sem1, sem2, sem3):
    wid = lax.axis_index("s") * NC + lax.axis_index("c")
    b = wid // 4
    base = (wid % 4) * PW

    i0 = (wid % 4) * 16
    d_gtx = pltpu.async_copy(gt_hbm.at[b, 0, pl.ds(i0, 16)], gtx_v, sem0)
    d_gty = pltpu.async_copy(gt_hbm.at[b, 1, pl.ds(i0, 16)], gty_v, sem1)
    d_r0 = pltpu.async_copy(ref_hbm.at[b, 0], ref_v.at[pl.ds(0, S)], sem2)
    d_r1 = pltpu.async_copy(ref_hbm.at[b, 1], ref_v.at[pl.ds(S, S)], sem3)
    d_r2 = pltpu.async_copy(ref_hbm.at[b, 2], ref_v.at[pl.ds(2 * S, S)], sem2)
    d_gtx.wait()
    d_gty.wait()
    d_r0.wait()
    d_r1.wait()
    d_r2.wait()

    lanes = lax.iota(jnp.int32, 16)
    rbase0 = b * P + base
    eb_lanes = ((lanes >> 3) << 15) + ((lanes & 7) << 7)

    def phase_a(ch, carry):
        def a_iter(tt, carry2):
            off = ch * CH + tt * 16
            sl = pl.ds(off, 16)
            ir = ch * 2 + (tt >> 2)
            jsl = pl.ds((tt & 3) * 16, 16)
            gx = (gtx_v[ir, jsl] + 1.0) * 63.0 / 2.0
            gy = (gty_v[ir, jsl] + 1.0) * 63.0 / 2.0
            fx = gx.astype(jnp.int32)
            fy = gy.astype(jnp.int32)
            fxf = fx.astype(jnp.float32)
            fyf = fy.astype(jnp.float32)

            cx1 = jnp.minimum(fx + 1, S - 1)
            cy1 = jnp.minimum(fy + 1, S - 1)
            dx = cx1 - fx
            dy = cy1 - fy

            ebase = ((rbase0 + off) >> 3) * 32768 + eb_lanes + fx
            e0 = ebase + ((fy >> 1) << 10) + ((fy & 1) << 6)
            e2 = ebase + ((cy1 >> 1) << 10) + ((cy1 & 1) << 6)
            e0_v[sl] = e0
            e1_v[sl] = e0 + dx
            e2_v[sl] = e2
            e3_v[sl] = e2 + dx
            q0_v[sl] = fy
            q1_v[sl] = fx
            q2_v[sl] = cy1
            q3_v[sl] = cx1

            wx0 = (fxf + 1.0) - gx
            wx1 = gx - fxf
            wy0 = (fyf + 1.0) - gy
            wy1 = gy - fyf
            mx = dx.astype(jnp.float32)
            my = dy.astype(jnp.float32)
            a0 = wy0 * wx0 * (mx * my)
            a1 = wy0 * wx1 * my
            a2 = wy1 * wx0 * mx
            a3 = wy1 * wx1

            fyb = fy + S
            fyc = fy + 2 * S
            cyb = cy1 + S
            cyc = cy1 + 2 * S
            r0c0 = plsc.load_gather(ref_v, [fy, fx])
            r0c1 = plsc.load_gather(ref_v, [fyb, fx])
            r0c2 = plsc.load_gather(ref_v, [fyc, fx])
            r1c0 = plsc.load_gather(ref_v, [fy, cx1])
            r1c1 = plsc.load_gather(ref_v, [fyb, cx1])
            r1c2 = plsc.load_gather(ref_v, [fyc, cx1])
            r2c0 = plsc.load_gather(ref_v, [cy1, fx])
            r2c1 = plsc.load_gather(ref_v, [cyb, fx])
            r2c2 = plsc.load_gather(ref_v, [cyc, fx])
            r3c0 = plsc.load_gather(ref_v, [cy1, cx1])
            r3c1 = plsc.load_gather(ref_v, [cyb, cx1])
            r3c2 = plsc.load_gather(ref_v, [cyc, cx1])

            osl = pl.ds((off >> 6) * 128 + (off & 63), 16)
            os0_v[osl] = a0 * r0c0 + a1 * r1c0 + a2 * r2c0 + a3 * r3c0
            os1_v[osl] = a0 * r0c1 + a1 * r1c1 + a2 * r2c1 + a3 * r3c1
            os2_v[osl] = a0 * r0c2 + a1 * r1c2 + a2 * r2c2 + a3 * r3c2
            return carry2

        lax.fori_loop(0, TPC, a_iter, 0)

        csl = pl.ds(ch * CH, CH)
        pltpu.async_copy(corr_hbm.at[e0_v.at[csl]], cb0_v.at[csl], sem0)
        pltpu.async_copy(corr_hbm.at[e1_v.at[csl]], cb1_v.at[csl], sem1)
        pltpu.async_copy(corr_hbm.at[e2_v.at[csl]], cb2_v.at[csl], sem2)
        pltpu.async_copy(corr_hbm.at[e3_v.at[csl]], cb3_v.at[csl], sem3)
        return carry

    lax.fori_loop(0, NCH, phase_a, 0)

    def phase_b(ch, carry):
        csl = pl.ds(ch * CH, CH)
        pltpu.make_async_copy(corr_hbm.at[e0_v.at[csl]], cb0_v.at[csl], sem0).wait()
        pltpu.make_async_copy(corr_hbm.at[e1_v.at[csl]], cb1_v.at[csl], sem1).wait()
        pltpu.make_async_copy(corr_hbm.at[e2_v.at[csl]], cb2_v.at[csl], sem2).wait()
        pltpu.make_async_copy(corr_hbm.at[e3_v.at[csl]], cb3_v.at[csl], sem3).wait()
        return carry

    lax.fori_loop(0, NCH, phase_b, 0)

    @plsc.parallel_loop(0, PW // 16)
    def phase_c(t):
        sl = pl.ds(t * 16, 16)
        fy = q0_v[sl]
        fx = q1_v[sl]
        cy1 = q2_v[sl]
        cx1 = q3_v[sl]
        mx = (cx1 - fx).astype(jnp.float32)
        my = (cy1 - fy).astype(jnp.float32)

        c00 = cb0_v[sl] * (mx * my)
        c01 = cb1_v[sl] * my
        c10 = cb2_v[sl] * mx
        c11 = cb3_v[sl]

        fyb = fy + S
        fyc = fy + 2 * S
        cyb = cy1 + S
        cyc = cy1 + 2 * S
        r0c0 = plsc.load_gather(ref_v, [fy, fx])
        r0c1 = plsc.load_gather(ref_v, [fyb, fx])
        r0c2 = plsc.load_gather(ref_v, [fyc, fx])
        r1c0 = plsc.load_gather(ref_v, [fy, cx1])
        r1c1 = plsc.load_gather(ref_v, [fyb, cx1])
        r1c2 = plsc.load_gather(ref_v, [fyc, cx1])
        r2c0 = plsc.load_gather(ref_v, [cy1, fx])
        r2c1 = plsc.load_gather(ref_v, [cyb, fx])
        r2c2 = plsc.load_gather(ref_v, [cyc, fx])
        r3c0 = plsc.load_gather(ref_v, [cy1, cx1])
        r3c1 = plsc.load_gather(ref_v, [cyb, cx1])
        r3c2 = plsc.load_gather(ref_v, [cyc, cx1])

        osl = pl.ds((t >> 2) * 128 + ((t & 3) << 4), 16)
        oc0_v[osl] = c00 * r0c0 + c01 * r1c0 + c10 * r2c0 + c11 * r3c0
        oc1_v[osl] = c00 * r0c1 + c01 * r1c1 + c10 * r2c1 + c11 * r3c1
        oc2_v[osl] = c00 * r0c2 + c01 * r1c2 + c10 * r2c2 + c11 * r3c2

    obase = (wid % 4) * 2048
    w0 = pltpu.async_copy(os0_v, ws_hbm.at[pl.ds((b * 3 + 0) * OPIX + obase, 2048)], sem0)
    w1 = pltpu.async_copy(os1_v, ws_hbm.at[pl.ds((b * 3 + 1) * OPIX + obase, 2048)], sem1)
    w2 = pltpu.async_copy(os2_v, ws_hbm.at[pl.ds((b * 3 + 2) * OPIX + obase, 2048)], sem2)
    w3 = pltpu.async_copy(oc0_v, wc_hbm.at[pl.ds((b * 3 + 0) * OPIX + obase, 2048)], sem3)
    w4 = pltpu.async_copy(oc1_v, wc_hbm.at[pl.ds((b * 3 + 1) * OPIX + obase, 2048)], sem0)
    w5 = pltpu.async_copy(oc2_v, wc_hbm.at[pl.ds((b * 3 + 2) * OPIX + obase, 2048)], sem1)
    w0.wait()
    w1.wait()
    w2.wait()
    w3.wait()
    w4.wait()
    w5.wait()


def _build_sc_call():
    mesh = plsc.VectorSubcoreMesh(core_axis_name="c", subcore_axis_name="s",
                                  num_cores=NC, num_subcores=NS)
    out = jax.ShapeDtypeStruct((B * 3 * OPIX,), jnp.float32)
    return pl.kernel(
        _sc_body,
        out_type=(out, out),
        mesh=mesh,
        scratch_types=[
            pltpu.VMEM((16, S), jnp.float32),
            pltpu.VMEM((16, S), jnp.float32),
            pltpu.VMEM((3 * S, S), jnp.float32),
            pltpu.VMEM((PW,), jnp.int32),
            pltpu.VMEM((PW,), jnp.int32),
            pltpu.VMEM((PW,), jnp.int32),
            pltpu.VMEM((PW,), jnp.int32),
            pltpu.VMEM((PW,), jnp.int32),
            pltpu.VMEM((PW,), jnp.int32),
            pltpu.VMEM((PW,), jnp.int32),
            pltpu.VMEM((PW,), jnp.int32),
            pltpu.VMEM((PW,), jnp.float32),
            pltpu.VMEM((PW,), jnp.float32),
            pltpu.VMEM((PW,), jnp.float32),
            pltpu.VMEM((PW,), jnp.float32),
            pltpu.VMEM((2048,), jnp.float32),
            pltpu.VMEM((2048,), jnp.float32),
            pltpu.VMEM((2048,), jnp.float32),
            pltpu.VMEM((2048,), jnp.float32),
            pltpu.VMEM((2048,), jnp.float32),
            pltpu.VMEM((2048,), jnp.float32),
            pltpu.SemaphoreType.DMA,
            pltpu.SemaphoreType.DMA,
            pltpu.SemaphoreType.DMA,
            pltpu.SemaphoreType.DMA,
        ],
        compiler_params=pltpu.CompilerParams(needs_layout_passes=False),
    )


def kernel(corr_m, gt_flow, vis_mask, scale_ref):
    del vis_mask
    corr_flat = (corr_m.reshape(B * P // 8, 8, P // 128, 128)
                 .transpose(0, 2, 1, 3)
                 .reshape(B * P * P))

    ws_p, wc_p = _build_sc_call()(corr_flat, gt_flow, scale_ref)
    warp_smpl = ws_p.reshape(B, 3, S, 128)[..., :S]
    warp_corr = wc_p.reshape(B, 3, S, 128)[..., :S]
    return (warp_smpl, warp_corr)


# speedup vs baseline: 1.1695x; 1.0012x over previous
import jax
import jax.numpy as jnp
from jax import lax
from jax.experimental import pallas as pl
from jax.experimental.pallas import tpu as pltpu
from jax.experimental.pallas import tpu_sc as plsc

B = 8
P = 4096
S = 64
NC = 2
NS = 16
NW = NC * NS
PW = (B * P) // NW
CH = 128
NCH = PW // CH
TPC = CH // 16
OPIX = S * 128


def _sc_body(corr_hbm, gt_hbm, ref_hbm, ws_hbm, wc_hbm,
             gtx_v, gty_v, ref_v,
             e0_v, e1_v, e2_v, e3_v,
             q0_v, q1_v, q2_v, q3_v,
             cb0_v, cb1_v, cb2_v, cb3_v,
             os0_v, os1_v, os2_v, oc0_v, oc1_v, oc2_v,
             sem0, sem1, sem2, sem3):
    wid = lax.axis_index("s") * NC + lax.axis_index("c")
    b = wid // 4
    base = (wid % 4) * PW

    i0 = (wid % 4) * 16
    d_gtx = pltpu.async_copy(gt_hbm.at[b, 0, pl.ds(i0, 16)], gtx_v, sem0)
    d_gty = pltpu.async_copy(gt_hbm.at[b, 1, pl.ds(i0, 16)], gty_v, sem1)
    d_r0 = pltpu.async_copy(ref_hbm.at[b, 0], ref_v.at[pl.ds(0, S)], sem2)
    d_r1 = pltpu.async_copy(ref_hbm.at[b, 1], ref_v.at[pl.ds(S, S)], sem3)
    d_r2 = pltpu.async_copy(ref_hbm.at[b, 2], ref_v.at[pl.ds(2 * S, S)], sem2)
    d_gtx.wait()
    d_gty.wait()
    d_r0.wait()
    d_r1.wait()
    d_r2.wait()

    lanes = lax.iota(jnp.int32, 16)
    rbase0 = b * P + base
    eb_lanes = ((lanes >> 3) << 15) + ((lanes & 7) << 7)

    def phase_a(ch, carry):
        @plsc.parallel_loop(0, TPC)
        def a_iter(tt):
            off = ch * CH + tt * 16
            sl = pl.ds(off, 16)
            ir = ch * 2 + (tt >> 2)
            jsl = pl.ds((tt & 3) * 16, 16)
            gx = (gtx_v[ir, jsl] + 1.0) * 63.0 / 2.0
            gy = (gty_v[ir, jsl] + 1.0) * 63.0 / 2.0
            fx = gx.astype(jnp.int32)
            fy = gy.astype(jnp.int32)
            fxf = fx.astype(jnp.float32)
            fyf = fy.astype(jnp.float32)

            cx1 = jnp.minimum(fx + 1, S - 1)
            cy1 = jnp.minimum(fy + 1, S - 1)
            dx = cx1 - fx
            dy = cy1 - fy

            ebase = ((rbase0 + off) >> 3) * 32768 + eb_lanes + fx
            e0 = ebase + ((fy >> 1) << 10) + ((fy & 1) << 6)
            e2 = ebase + ((cy1 >> 1) << 10) + ((cy1 & 1) << 6)
            e0_v[sl] = e0
            e1_v[sl] = e0 + dx
            e2_v[sl] = e2
            e3_v[sl] = e2 + dx
            q0_v[sl] = fy
            q1_v[sl] = fx
            q2_v[sl] = cy1
            q3_v[sl] = cx1

            wx0 = (fxf + 1.0) - gx
            wx1 = gx - fxf
            wy0 = (fyf + 1.0) - gy
            wy1 = gy - fyf
            mx = dx.astype(jnp.float32)
            my = dy.astype(jnp.float32)
            a0 = wy0 * wx0 * (mx * my)
            a1 = wy0 * wx1 * my
            a2 = wy1 * wx0 * mx
            a3 = wy1 * wx1

            fyb = fy + S
            fyc = fy + 2 * S
            cyb = cy1 + S
            cyc = cy1 + 2 * S
            r0c0 = plsc.load_gather(ref_v, [fy, fx])
            r0c1 = plsc.load_gather(ref_v, [fyb, fx])
            r0c2 = plsc.load_gather(ref_v, [fyc, fx])
            r1c0 = plsc.load_gather(ref_v, [fy, cx1])
            r1c1 = plsc.load_gather(ref_v, [fyb, cx1])
            r1c2 = plsc.load_gather(ref_v, [fyc, cx1])
            r2c0 = plsc.load_gather(ref_v, [cy1, fx])
            r2c1 = plsc.load_gather(ref_v, [cyb, fx])
            r2c2 = plsc.load_gather(ref_v, [cyc, fx])
            r3c0 = plsc.load_gather(ref_v, [cy1, cx1])
            r3c1 = plsc.load_gather(ref_v, [cyb, cx1])
            r3c2 = plsc.load_gather(ref_v, [cyc, cx1])

            osl = pl.ds((off >> 6) * 128 + (off & 63), 16)
            os0_v[osl] = a0 * r0c0 + a1 * r1c0 + a2 * r2c0 + a3 * r3c0
            os1_v[osl] = a0 * r0c1 + a1 * r1c1 + a2 * r2c1 + a3 * r3c1
            os2_v[osl] = a0 * r0c2 + a1 * r1c2 + a2 * r2c2 + a3 * r3c2

        csl = pl.ds(ch * CH, CH)
        pltpu.async_copy(corr_hbm.at[e0_v.at[csl]], cb0_v.at[csl], sem0)
        pltpu.async_copy(corr_hbm.at[e1_v.at[csl]], cb1_v.at[csl], sem1)
        pltpu.async_copy(corr_hbm.at[e2_v.at[csl]], cb2_v.at[csl], sem2)
        pltpu.async_copy(corr_hbm.at[e3_v.at[csl]], cb3_v.at[csl], sem3)
        return carry

    lax.fori_loop(0, NCH, phase_a, 0)

    def phase_b(ch, carry):
        csl = pl.ds(ch * CH, CH)
        pltpu.make_async_copy(corr_hbm.at[e0_v.at[csl]], cb0_v.at[csl], sem0).wait()
        pltpu.make_async_copy(corr_hbm.at[e1_v.at[csl]], cb1_v.at[csl], sem1).wait()
        pltpu.make_async_copy(corr_hbm.at[e2_v.at[csl]], cb2_v.at[csl], sem2).wait()
        pltpu.make_async_copy(corr_hbm.at[e3_v.at[csl]], cb3_v.at[csl], sem3).wait()
        return carry

    lax.fori_loop(0, NCH, phase_b, 0)

    @plsc.parallel_loop(0, PW // 16)
    def phase_c(t):
        sl = pl.ds(t * 16, 16)
        fy = q0_v[sl]
        fx = q1_v[sl]
        cy1 = q2_v[sl]
        cx1 = q3_v[sl]
        mx = (cx1 - fx).astype(jnp.float32)
        my = (cy1 - fy).astype(jnp.float32)

        c00 = cb0_v[sl] * (mx * my)
        c01 = cb1_v[sl] * my
        c10 = cb2_v[sl] * mx
        c11 = cb3_v[sl]

        fyb = fy + S
        fyc = fy + 2 * S
        cyb = cy1 + S
        cyc = cy1 + 2 * S
        r0c0 = plsc.load_gather(ref_v, [fy, fx])
        r0c1 = plsc.load_gather(ref_v, [fyb, fx])
        r0c2 = plsc.load_gather(ref_v, [fyc, fx])
        r1c0 = plsc.load_gather(ref_v, [fy, cx1])
        r1c1 = plsc.load_gather(ref_v, [fyb, cx1])
        r1c2 = plsc.load_gather(ref_v, [fyc, cx1])
        r2c0 = plsc.load_gather(ref_v, [cy1, fx])
        r2c1 = plsc.load_gather(ref_v, [cyb, fx])
        r2c2 = plsc.load_gather(ref_v, [cyc, fx])
        r3c0 = plsc.load_gather(ref_v, [cy1, cx1])
        r3c1 = plsc.load_gather(ref_v, [cyb, cx1])
        r3c2 = plsc.load_gather(ref_v, [cyc, cx1])

        osl = pl.ds((t >> 2) * 128 + ((t & 3) << 4), 16)
        oc0_v[osl] = c00 * r0c0 + c01 * r1c0 + c10 * r2c0 + c11 * r3c0
        oc1_v[osl] = c00 * r0c1 + c01 * r1c1 + c10 * r2c1 + c11 * r3c1
        oc2_v[osl] = c00 * r0c2 + c01 * r1c2 + c10 * r2c2 + c11 * r3c2

    obase = (wid % 4) * 2048
    w0 = pltpu.async_copy(os0_v, ws_hbm.at[pl.ds((b * 3 + 0) * OPIX + obase, 2048)], sem0)
    w1 = pltpu.async_copy(os1_v, ws_hbm.at[pl.ds((b * 3 + 1) * OPIX + obase, 2048)], sem1)
    w2 = pltpu.async_copy(os2_v, ws_hbm.at[pl.ds((b * 3 + 2) * OPIX + obase, 2048)], sem2)
    w3 = pltpu.async_copy(oc0_v, wc_hbm.at[pl.ds((b * 3 + 0) * OPIX + obase, 2048)], sem3)
    w4 = pltpu.async_copy(oc1_v, wc_hbm.at[pl.ds((b * 3 + 1) * OPIX + obase, 2048)], sem0)
    w5 = pltpu.async_copy(oc2_v, wc_hbm.at[pl.ds((b * 3 + 2) * OPIX + obase, 2048)], sem1)
    w0.wait()
    w1.wait()
    w2.wait()
    w3.wait()
    w4.wait()
    w5.wait()


def _build_sc_call():
    mesh = plsc.VectorSubcoreMesh(core_axis_name="c", subcore_axis_name="s",
                                  num_cores=NC, num_subcores=NS)
    out = jax.ShapeDtypeStruct((B * 3 * OPIX,), jnp.float32)
    return pl.kernel(
        _sc_body,
        out_type=(out, out),
        mesh=mesh,
        scratch_types=[
            pltpu.VMEM((16, S), jnp.float32),
            pltpu.VMEM((16, S), jnp.float32),
            pltpu.VMEM((3 * S, S), jnp.float32),
            pltpu.VMEM((PW,), jnp.int32),
            pltpu.VMEM((PW,), jnp.int32),
            pltpu.VMEM((PW,), jnp.int32),
            pltpu.VMEM((PW,), jnp.int32),
            pltpu.VMEM((PW,), jnp.int32),
            pltpu.VMEM((PW,), jnp.int32),
            pltpu.VMEM((PW,), jnp.int32),
            pltpu.VMEM((PW,), jnp.int32),
            pltpu.VMEM((PW,), jnp.float32),
            pltpu.VMEM((PW,), jnp.float32),
            pltpu.VMEM((PW,), jnp.float32),
            pltpu.VMEM((PW,), jnp.float32),
            pltpu.VMEM((2048,), jnp.float32),
            pltpu.VMEM((2048,), jnp.float32),
            pltpu.VMEM((2048,), jnp.float32),
            pltpu.VMEM((2048,), jnp.float32),
            pltpu.VMEM((2048,), jnp.float32),
            pltpu.VMEM((2048,), jnp.float32),
            pltpu.SemaphoreType.DMA,
            pltpu.SemaphoreType.DMA,
            pltpu.SemaphoreType.DMA,
            pltpu.SemaphoreType.DMA,
        ],
        compiler_params=pltpu.CompilerParams(needs_layout_passes=False),
    )


def kernel(corr_m, gt_flow, vis_mask, scale_ref):
    del vis_mask
    corr_flat = (corr_m.reshape(B * P // 8, 8, P // 128, 128)
                 .transpose(0, 2, 1, 3)
                 .reshape(B * P * P))

    ws_p, wc_p = _build_sc_call()(corr_flat, gt_flow, scale_ref)
    warp_smpl = ws_p.reshape(B, 3, S, 128)[..., :S]
    warp_corr = wc_p.reshape(B, 3, S, 128)[..., :S]
    return (warp_smpl, warp_corr)
